# baseline (speedup 1.0000x reference)
import jax
import jax.numpy as jnp
from jax.experimental import pallas as pl
from jax.experimental.pallas import tpu as pltpu

_NUM_CLASSES = 5
_MIN_MIXED = 128
_GROUPS_PER_STEP = 2


def _group_loss(sca_ref, scb_ref, lca_ref, lcb_ref,
                srow_ref, srev_ref, lrow_ref, lrev_ref, q, g):
    h = g // 2
    qh = q * h
    c1 = -1.4426950408889634
    c2 = 0.5 / 0.6931471805599453
    ln2 = 0.6931471805599453

    lrow_full = lrow_ref[q]
    srow_full = srow_ref[q]
    sumsq = jnp.zeros((), jnp.float32)
    lin = jnp.zeros((), jnp.float32)
    for a in range(_NUM_CLASSES):
        cnt = jnp.sum(jnp.where(lrow_full == a, 1.0, 0.0))
        sumsq += cnt * cnt
        lin += cnt * jnp.sum(
            srow_full * jnp.sign(lrow_full - a).astype(jnp.float32))
    n_pairs = (float(g * g) - sumsq) * 0.5

    def t_of(a):
        return jnp.log2(1.0 + jnp.exp2(a * c1)) + c2 * a

    sca = sca_ref[qh:qh + h]
    scb = scb_ref[qh:qh + h]
    lca = lca_ref[qh:qh + h]
    lcb = lcb_ref[qh:qh + h]
    srow = srow_full
    srev = srev_ref[q]
    lrow = lrow_full
    lrev = lrev_ref[q]

    sums = []

    def emit_top(r0, r1, c0, c1_):
        d = sca[r0:r1] - srow[:, c0:c1_]
        m = lca[r0:r1] != lrow[:, c0:c1_]
        sums.append(jnp.sum(jnp.where(m, t_of(jnp.abs(d)), 0.0)))

    def emit_bottom(r0, r1, c0, c1_):
        d = scb[r0:r1] - srev[:, c0:c1_]
        m = lcb[r0:r1] != lrev[:, c0:c1_]
        sums.append(jnp.sum(jnp.where(m, t_of(jnp.abs(d)), 0.0)))

    def emit_mixed(r0, r1, c0, c1_):
        shp = (r1 - r0, c1_ - c0)
        rr = r0 + jax.lax.broadcasted_iota(jnp.int32, shp, 0)
        jj = c0 + jax.lax.broadcasted_iota(jnp.int32, shp, 1)
        top = jj < rr
        d = jnp.where(top, sca[r0:r1] - srow[:, c0:c1_],
                      scb[r0:r1] - srev[:, c0:c1_])
        m = jnp.logical_or(
            jnp.logical_and(top, lca[r0:r1] != lrow[:, c0:c1_]),
            jnp.logical_and(jnp.logical_not(top),
                            lcb[r0:r1] != lrev[:, c0:c1_]))
        sums.append(jnp.sum(jnp.where(m, t_of(jnp.abs(d)), 0.0)))

    def emit(r0, r1, c0, c1_):
        if c1_ <= r0:
            emit_top(r0, r1, c0, c1_)
        elif c0 >= r1 - 1:
            emit_bottom(r0, r1, c0, c1_)
        elif r1 - r0 <= _MIN_MIXED:
            emit_mixed(r0, r1, c0, c1_)
        else:
            rm = (r0 + r1) // 2
            cm = (c0 + c1_) // 2
            emit(r0, rm, c0, cm)
            emit(r0, rm, cm, c1_)
            emit(rm, r1, c0, cm)
            emit(rm, r1, cm, c1_)

    emit(0, h, 0, h)
    emit(0, h, h, g)
    tot = sums[0]
    for s_part in sums[1:]:
        tot = tot + s_part

    safe_n = jnp.where(n_pairs > 0, n_pairs, 1.0)
    loss = (ln2 * tot - 0.5 * lin) / safe_n
    return loss, n_pairs


def _rank_loss_kernel(sca_ref, scb_ref, lca_ref, lcb_ref,
                      srow_ref, srev_ref, lrow_ref, lrev_ref,
                      out_ref, acc_ref):
    p = pl.program_id(0)
    np_ = pl.num_programs(0)
    g = lrow_ref.shape[2]
    h = g // 2

    @pl.when(p == 0)
    def _init_totals():
        acc_ref[0] = 0.0
        acc_ref[1] = 0.0

    for q in range(_GROUPS_PER_STEP):
        loss, n_pairs = _group_loss(
            sca_ref, scb_ref, lca_ref, lcb_ref,
            srow_ref, srev_ref, lrow_ref, lrev_ref, q, g)
        acc_ref[0] += jnp.where(n_pairs > 0, loss, 0.0)
        acc_ref[1] += jnp.where(n_pairs > 0, 1.0, 0.0)

    @pl.when(p == np_ - 1)
    def _finalize_output():
        count = acc_ref[1]
        safe_c = jnp.where(count > 0, count, 1.0)
        out_ref[0, 0] = jnp.where(count > 0, acc_ref[0] / safe_c, 0.0)


def kernel(scores, labels, group_sizes):
    scores = scores.reshape(-1)
    labels = labels.reshape(-1)
    n = scores.shape[0]
    num_groups = group_sizes.shape[0]
    g = n // num_groups
    h = g // 2
    gps = _GROUPS_PER_STEP
    steps = num_groups // gps

    s2 = scores.reshape(num_groups, g)
    l2 = labels.reshape(num_groups, g)
    sca = s2[:, :h].reshape(num_groups * h, 1)
    scb = s2[:, :h - 1:-1].reshape(num_groups * h, 1)
    lca = l2[:, :h].reshape(num_groups * h, 1)
    lcb = l2[:, :h - 1:-1].reshape(num_groups * h, 1)
    srow = s2.reshape(num_groups, 1, g)
    srev = s2[:, ::-1].reshape(num_groups, 1, g)
    lrow = l2.reshape(num_groups, 1, g)
    lrev = l2[:, ::-1].reshape(num_groups, 1, g)

    col = pl.BlockSpec((gps * h, 1), lambda p: (p, 0))
    row = pl.BlockSpec((gps, 1, g), lambda p: (p, 0, 0))

    out = pl.pallas_call(
        _rank_loss_kernel,
        grid=(steps,),
        in_specs=[col, col, col, col, row, row, row, row],
        out_specs=pl.BlockSpec(memory_space=pltpu.SMEM),
        out_shape=jax.ShapeDtypeStruct((1, 1), jnp.float32),
        scratch_shapes=[pltpu.SMEM((2,), jnp.float32)],
    )(sca, scb, lca, lcb, srow, srev, lrow, lrev)
    return out[0, 0]


# speedup vs baseline: 1.0094x; 1.0094x over previous
import jax
import jax.numpy as jnp
from jax.experimental import pallas as pl
from jax.experimental.pallas import tpu as pltpu

_NUM_CLASSES = 5
_MIN_MIXED = 128
_GROUPS_PER_STEP = 1


def _group_loss(sca_ref, scb_ref, lca_ref, lcb_ref,
                srow_ref, srev_ref, lrow_ref, lrev_ref, q, g):
    h = g // 2
    qh = q * h
    c1 = -1.4426950408889634
    c2 = 0.5 / 0.6931471805599453
    ln2 = 0.6931471805599453

    lrow_full = lrow_ref[q]
    srow_full = srow_ref[q]
    sumsq = jnp.zeros((), jnp.float32)
    lin = jnp.zeros((), jnp.float32)
    for a in range(_NUM_CLASSES):
        cnt = jnp.sum(jnp.where(lrow_full == a, 1.0, 0.0))
        sumsq += cnt * cnt
        lin += cnt * jnp.sum(
            srow_full * jnp.sign(lrow_full - a).astype(jnp.float32))
    n_pairs = (float(g * g) - sumsq) * 0.5

    def t_of(a):
        return jnp.log2(1.0 + jnp.exp2(a * c1)) + c2 * a

    sca = sca_ref[qh:qh + h]
    scb = scb_ref[qh:qh + h]
    lca = lca_ref[qh:qh + h]
    lcb = lcb_ref[qh:qh + h]
    srow = srow_full
    srev = srev_ref[q]
    lrow = lrow_full
    lrev = lrev_ref[q]

    sums = []

    def emit_top(r0, r1, c0, c1_):
        d = sca[r0:r1] - srow[:, c0:c1_]
        m = lca[r0:r1] != lrow[:, c0:c1_]
        sums.append(jnp.sum(jnp.where(m, t_of(jnp.abs(d)), 0.0)))

    def emit_bottom(r0, r1, c0, c1_):
        d = scb[r0:r1] - srev[:, c0:c1_]
        m = lcb[r0:r1] != lrev[:, c0:c1_]
        sums.append(jnp.sum(jnp.where(m, t_of(jnp.abs(d)), 0.0)))

    def emit_mixed(r0, r1, c0, c1_):
        shp = (r1 - r0, c1_ - c0)
        rr = r0 + jax.lax.broadcasted_iota(jnp.int32, shp, 0)
        jj = c0 + jax.lax.broadcasted_iota(jnp.int32, shp, 1)
        top = jj < rr
        d = jnp.where(top, sca[r0:r1] - srow[:, c0:c1_],
                      scb[r0:r1] - srev[:, c0:c1_])
        m = jnp.logical_or(
            jnp.logical_and(top, lca[r0:r1] != lrow[:, c0:c1_]),
            jnp.logical_and(jnp.logical_not(top),
                            lcb[r0:r1] != lrev[:, c0:c1_]))
        sums.append(jnp.sum(jnp.where(m, t_of(jnp.abs(d)), 0.0)))

    def emit(r0, r1, c0, c1_):
        if c1_ <= r0:
            emit_top(r0, r1, c0, c1_)
        elif c0 >= r1 - 1:
            emit_bottom(r0, r1, c0, c1_)
        elif r1 - r0 <= _MIN_MIXED:
            emit_mixed(r0, r1, c0, c1_)
        else:
            rm = (r0 + r1) // 2
            cm = (c0 + c1_) // 2
            emit(r0, rm, c0, cm)
            emit(r0, rm, cm, c1_)
            emit(rm, r1, c0, cm)
            emit(rm, r1, cm, c1_)

    emit(0, h, 0, h)
    emit(0, h, h, g)
    tot = sums[0]
    for s_part in sums[1:]:
        tot = tot + s_part

    safe_n = jnp.where(n_pairs > 0, n_pairs, 1.0)
    loss = (ln2 * tot - 0.5 * lin) / safe_n
    return loss, n_pairs


def _rank_loss_kernel(sca_ref, scb_ref, lca_ref, lcb_ref,
                      srow_ref, srev_ref, lrow_ref, lrev_ref,
                      out_ref, acc_ref):
    p = pl.program_id(0)
    np_ = pl.num_programs(0)
    g = lrow_ref.shape[2]
    h = g // 2

    @pl.when(p == 0)
    def _init_totals():
        acc_ref[0] = 0.0
        acc_ref[1] = 0.0

    for q in range(_GROUPS_PER_STEP):
        loss, n_pairs = _group_loss(
            sca_ref, scb_ref, lca_ref, lcb_ref,
            srow_ref, srev_ref, lrow_ref, lrev_ref, q, g)
        acc_ref[0] += jnp.where(n_pairs > 0, loss, 0.0)
        acc_ref[1] += jnp.where(n_pairs > 0, 1.0, 0.0)

    @pl.when(p == np_ - 1)
    def _finalize_output():
        count = acc_ref[1]
        safe_c = jnp.where(count > 0, count, 1.0)
        out_ref[0, 0] = jnp.where(count > 0, acc_ref[0] / safe_c, 0.0)


def kernel(scores, labels, group_sizes):
    scores = scores.reshape(-1)
    labels = labels.reshape(-1)
    n = scores.shape[0]
    num_groups = group_sizes.shape[0]
    g = n // num_groups
    h = g // 2
    gps = _GROUPS_PER_STEP
    steps = num_groups // gps

    s2 = scores.reshape(num_groups, g)
    l2 = labels.reshape(num_groups, g)
    sca = s2[:, :h].reshape(num_groups * h, 1)
    scb = s2[:, :h - 1:-1].reshape(num_groups * h, 1)
    lca = l2[:, :h].reshape(num_groups * h, 1)
    lcb = l2[:, :h - 1:-1].reshape(num_groups * h, 1)
    srow = s2.reshape(num_groups, 1, g)
    srev = s2[:, ::-1].reshape(num_groups, 1, g)
    lrow = l2.reshape(num_groups, 1, g)
    lrev = l2[:, ::-1].reshape(num_groups, 1, g)

    col = pl.BlockSpec((gps * h, 1), lambda p: (p, 0))
    row = pl.BlockSpec((gps, 1, g), lambda p: (p, 0, 0))

    out = pl.pallas_call(
        _rank_loss_kernel,
        grid=(steps,),
        in_specs=[col, col, col, col, row, row, row, row],
        out_specs=pl.BlockSpec(memory_space=pltpu.SMEM),
        out_shape=jax.ShapeDtypeStruct((1, 1), jnp.float32),
        scratch_shapes=[pltpu.SMEM((2,), jnp.float32)],
    )(sca, scb, lca, lcb, srow, srev, lrow, lrev)
    return out[0, 0]
